# baseline (speedup 1.0000x reference)
import functools

import jax
import jax.numpy as jnp
import numpy as np
from jax import lax
from jax.experimental import pallas as pl
from jax.experimental.pallas import tpu as pltpu
from jax.experimental.pallas import tpu_sc as plsc

_B, _S, _V = 32, 200, 1000
_NBI = _S - 1
_NTRI = _S - 2
_NCH = 13

_DUMP = 1_040_000
_SH_WORDS = 1_048_576

_mesh = plsc.VectorSubcoreMesh(core_axis_name="c", subcore_axis_name="s",
                               num_cores=1)


def _conf_hist_body(lg_ref, toks_ref, conf_ref, counts_ref, pres_ref):
    i = pl.program_id(0)
    x = lg_ref[...]
    m = jnp.max(x, axis=1, keepdims=True)
    s = jnp.sum(jnp.exp(x - m), axis=1)
    part = jnp.sum(1.0 / s)

    tok8 = toks_ref[pl.ds(i * 16, 16), :]
    iota_v = lax.broadcasted_iota(jnp.int32, (1, 1, _V), 2)
    cmp = tok8[:, :, None] == iota_v
    cnt = jnp.sum(cmp.astype(jnp.float32), axis=(0, 1))
    pres_ref[pl.ds(i * 16, 16), :] = jnp.any(cmp, axis=1).astype(jnp.float32)

    @pl.when(i == 0)
    def _():
        conf_ref[...] = jnp.zeros((1, 1), jnp.float32)
        counts_ref[...] = jnp.zeros((1, _V), jnp.float32)

    conf_ref[...] += jnp.full((1, 1), part)
    counts_ref[...] += cnt[None, :]


def _stats_body(counts_ref, pres_ref, pbi_ref, ptri_ref, ppr_ref, conf_ref,
                out_ref):
    counts = counts_ref[0, :]
    total = jnp.sum(counts)
    probs = counts / (total + 1e-08)
    entropy = -jnp.sum(jnp.where(probs > 0, probs * jnp.log(probs + 1e-08), 0.0))
    token_entropy = 1.0 - entropy / np.log(_V)
    distinct1 = jnp.sum((counts > 0).astype(jnp.float32))

    pres = pres_ref[...]
    inter = lax.dot_general(pres, pres, (((1,), (1,)), ((), ())),
                            preferred_element_type=jnp.float32)
    ru = jnp.sum(pres, axis=1)
    r_i = lax.broadcasted_iota(jnp.int32, (_B, _B), 0)
    c_i = lax.broadcasted_iota(jnp.int32, (_B, _B), 1)
    selmask = ((r_i < 10) & (r_i != c_i)).astype(jnp.float32)
    overlaps = inter / jnp.maximum(ru, 1.0)[:, None]
    self_bleu = jnp.sum(overlaps * selmask) / (10 * (_B - 1))

    u_bi = jnp.sum(pbi_ref[...]).astype(jnp.float32)
    u_tri = jnp.sum(ptri_ref[...]).astype(jnp.float32)
    u_pr = jnp.sum(ppr_ref[...]).astype(jnp.float32)

    repetition = 1.0 - u_pr / (_B * _NTRI)
    d1 = distinct1 / (_B * _S)
    d2 = u_bi / (_B * _NBI)
    d3 = u_tri / (_B * _NTRI)
    ngram_diversity = ((1.0 - d1) + (1.0 - d2) + (1.0 - d3)) / 3.0

    avg_conf = jnp.sum(conf_ref[...]) / (_B * _S)
    overconfidence = jnp.maximum(avg_conf - 0.85, 0.0) * 2.0

    total_loss = (0.25 * ngram_diversity + 0.2 * token_entropy + 0.2 * self_bleu
                  + 0.2 * repetition + 0.15 * overconfidence)

    out_ref[...] = jnp.stack([ngram_diversity, token_entropy, self_bleu,
                              repetition, overconfidence, total_loss])[None, :]


def _prep_body(toks_ref, bik_ref, t2k_ref):
    toks = toks_ref[...]
    bi = toks[:, :-1] * _V + toks[:, 1:]
    bik_ref[...] = jnp.concatenate(
        [bi, jnp.full((_B, 9), _DUMP, jnp.int32)], axis=1)
    t2k_ref[...] = jnp.concatenate(
        [toks[:, 2:], jnp.zeros((_B, 10), jnp.int32)], axis=1)


def _v16(x):
    return jnp.full((16,), x, jnp.int32)


@functools.partial(
    pl.kernel, mesh=_mesh,
    out_type=[jax.ShapeDtypeStruct((_B, 16), jnp.int32),
              jax.ShapeDtypeStruct((_B, 16), jnp.int32),
              jax.ShapeDtypeStruct((_B, 16), jnp.int32)],
    scratch_types=[pltpu.VMEM_SHARED((_SH_WORDS,), jnp.int32)]
                  + [pltpu.VMEM((_NCH, 16), jnp.int32) for _ in range(10)]
                  + [pltpu.VMEM((16,), jnp.int32), pltpu.SemaphoreType.DMA])
def _sc_distinct(bik_hbm, t2k_hbm, pbi_hbm, ptri_hbm, ppr_hbm,
                 shared,
                 key0, key1, t20, t21, ki0, ki1,
                 v0, v1, g0, g1, acc_v, sem):
    w = lax.axis_index("s")
    rows = (2 * w, 2 * w + 1)
    keys = (key0, key1)
    t2s = (t20, t21)
    kis = (ki0, ki1)
    vs = (v0, v1)
    gs = (g0, g1)
    iota = lax.iota(jnp.int32, 16)
    one = jnp.ones((16,), jnp.int32)
    zero = jnp.zeros((16,), jnp.int32)

    def scatter_all():
        cps = []
        for i in (0, 1):
            for c in range(_NCH):
                cps.append(pltpu.async_copy(vs[i].at[c],
                                            shared.at[kis[i].at[c]], sem))
        for cp in cps:
            cp.wait()
        plsc.subcore_barrier()

    def gather_all():
        cps = []
        for i in (0, 1):
            for c in range(_NCH):
                cps.append(pltpu.async_copy(shared.at[kis[i].at[c]],
                                            gs[i].at[c], sem))
        for cp in cps:
            cp.wait()

    for i in (0, 1):
        pltpu.sync_copy(bik_hbm.at[rows[i]], keys[i])
        pltpu.sync_copy(t2k_hbm.at[rows[i]], t2s[i])

    for i in (0, 1):
        for c in range(_NCH):
            kis[i][c, :] = keys[i][c, :]
            vs[i][c, :] = rows[i] * _NBI + c * 16 + iota
    scatter_all()
    gather_all()
    for i in (0, 1):
        acc = zero
        for c in range(_NCH):
            s_c = c * 16 + iota
            g_c = gs[i][c, :]
            j_c = rows[i] * _NBI + s_c
            acc = acc + jnp.where((s_c < _NBI) & (g_c == j_c), one, zero)
            kis[i][c, :] = jnp.where(s_c < _NTRI,
                                     g_c * 8 + (t2s[i][c, :] >> 7), _v16(_DUMP))
            vs[i][c, :] = rows[i] * _NTRI + s_c
        acc_v[...] = acc
        pltpu.sync_copy(acc_v, pbi_hbm.at[rows[i]])
    plsc.subcore_barrier()

    scatter_all()
    gather_all()
    for i in (0, 1):
        for c in range(_NCH):
            s_c = c * 16 + iota
            q_c = gs[i][c, :]
            kis[i][c, :] = jnp.where(s_c < _NTRI,
                                     q_c * 128 + (t2s[i][c, :] & 127), _v16(_DUMP))
    plsc.subcore_barrier()

    scatter_all()
    gather_all()
    for i in (0, 1):
        acc = zero
        for c in range(_NCH):
            s_c = c * 16 + iota
            h_c = gs[i][c, :]
            j3_c = rows[i] * _NTRI + s_c
            acc = acc + jnp.where((s_c < _NTRI) & (h_c == j3_c), one, zero)
            kis[i][c, :] = jnp.where(s_c < _NTRI,
                                     rows[i] * (_B * _NTRI) + h_c, _v16(_DUMP))
        acc_v[...] = acc
        pltpu.sync_copy(acc_v, ptri_hbm.at[rows[i]])
    plsc.subcore_barrier()

    scatter_all()
    gather_all()
    for i in (0, 1):
        acc = zero
        for c in range(_NCH):
            s_c = c * 16 + iota
            j3_c = rows[i] * _NTRI + s_c
            acc = acc + jnp.where((s_c < _NTRI) & (gs[i][c, :] == j3_c), one, zero)
        acc_v[...] = acc
        pltpu.sync_copy(acc_v, ppr_hbm.at[rows[i]])


@jax.jit
def _run(toks, logits):
    toks = toks.astype(jnp.int32)
    lg2 = logits.reshape(_B * _S, _V)
    rows = 3200

    bik, t2k = pl.pallas_call(
        _prep_body,
        out_shape=[jax.ShapeDtypeStruct((_B, 208), jnp.int32),
                   jax.ShapeDtypeStruct((_B, 208), jnp.int32)],
    )(toks)
    bik3 = bik.reshape(_B, _NCH, 16)
    t2k3 = t2k.reshape(_B, _NCH, 16)

    part_bi, part_tri, part_pr = _sc_distinct(bik3, t2k3)

    conf, counts, pres = pl.pallas_call(
        _conf_hist_body,
        grid=(_B * _S // rows,),
        in_specs=[pl.BlockSpec((rows, _V), lambda i: (i, 0)),
                  pl.BlockSpec((_B, _S), lambda i: (0, 0))],
        out_specs=[pl.BlockSpec((1, 1), lambda i: (0, 0)),
                   pl.BlockSpec((1, _V), lambda i: (0, 0)),
                   pl.BlockSpec((_B, _V), lambda i: (0, 0))],
        out_shape=[jax.ShapeDtypeStruct((1, 1), jnp.float32),
                   jax.ShapeDtypeStruct((1, _V), jnp.float32),
                   jax.ShapeDtypeStruct((_B, _V), jnp.float32)],
    )(lg2, toks)

    out = pl.pallas_call(
        _stats_body,
        out_shape=jax.ShapeDtypeStruct((1, 6), jnp.float32),
    )(counts, pres, part_bi, part_tri, part_pr, conf)
    return out.reshape(6)


def kernel(generated_tokens, generated_logits, vocab_size):
    return _run(generated_tokens, generated_logits)


# speedup vs baseline: 1.0308x; 1.0308x over previous
import functools

import jax
import jax.numpy as jnp
import numpy as np
from jax import lax
from jax.experimental import pallas as pl
from jax.experimental.pallas import tpu as pltpu
from jax.experimental.pallas import tpu_sc as plsc

_B, _S, _V = 32, 200, 1000
_NBI = _S - 1
_NTRI = _S - 2
_NCH = 13

_DUMP = 1_040_000
_SH_WORDS = 1_048_576

_mesh = plsc.VectorSubcoreMesh(core_axis_name="c", subcore_axis_name="s",
                               num_cores=1)


def _conf_hist_body(lg_ref, toks_ref, conf_ref, counts_ref, pres_ref):
    i = pl.program_id(0)
    x = lg_ref[...]
    m = jnp.max(x, axis=1, keepdims=True)
    s = jnp.sum(jnp.exp(x - m), axis=1)
    part = jnp.sum(1.0 / s)

    tok8 = toks_ref[pl.ds(i * 8, 8), :]
    iota_v = lax.broadcasted_iota(jnp.int32, (1, 1, _V), 2)
    cmp = tok8[:, :, None] == iota_v
    cnt = jnp.sum(cmp.astype(jnp.float32), axis=(0, 1))
    pres_ref[pl.ds(i * 8, 8), :] = jnp.any(cmp, axis=1).astype(jnp.float32)

    @pl.when(i == 0)
    def _():
        conf_ref[...] = jnp.zeros((1, 1), jnp.float32)
        counts_ref[...] = jnp.zeros((1, _V), jnp.float32)

    conf_ref[...] += jnp.full((1, 1), part)
    counts_ref[...] += cnt[None, :]


def _stats_body(counts_ref, pres_ref, pbi_ref, ptri_ref, ppr_ref, conf_ref,
                out_ref):
    counts = counts_ref[0, :]
    total = jnp.sum(counts)
    probs = counts / (total + 1e-08)
    entropy = -jnp.sum(jnp.where(probs > 0, probs * jnp.log(probs + 1e-08), 0.0))
    token_entropy = 1.0 - entropy / np.log(_V)
    distinct1 = jnp.sum((counts > 0).astype(jnp.float32))

    pres = pres_ref[...]
    inter = lax.dot_general(pres, pres, (((1,), (1,)), ((), ())),
                            preferred_element_type=jnp.float32)
    ru = jnp.sum(pres, axis=1)
    r_i = lax.broadcasted_iota(jnp.int32, (_B, _B), 0)
    c_i = lax.broadcasted_iota(jnp.int32, (_B, _B), 1)
    selmask = ((r_i < 10) & (r_i != c_i)).astype(jnp.float32)
    overlaps = inter / jnp.maximum(ru, 1.0)[:, None]
    self_bleu = jnp.sum(overlaps * selmask) / (10 * (_B - 1))

    u_bi = jnp.sum(pbi_ref[...]).astype(jnp.float32)
    u_tri = jnp.sum(ptri_ref[...]).astype(jnp.float32)
    u_pr = jnp.sum(ppr_ref[...]).astype(jnp.float32)

    repetition = 1.0 - u_pr / (_B * _NTRI)
    d1 = distinct1 / (_B * _S)
    d2 = u_bi / (_B * _NBI)
    d3 = u_tri / (_B * _NTRI)
    ngram_diversity = ((1.0 - d1) + (1.0 - d2) + (1.0 - d3)) / 3.0

    avg_conf = jnp.sum(conf_ref[...]) / (_B * _S)
    overconfidence = jnp.maximum(avg_conf - 0.85, 0.0) * 2.0

    total_loss = (0.25 * ngram_diversity + 0.2 * token_entropy + 0.2 * self_bleu
                  + 0.2 * repetition + 0.15 * overconfidence)

    out_ref[...] = jnp.stack([ngram_diversity, token_entropy, self_bleu,
                              repetition, overconfidence, total_loss])[None, :]


def _prep_body(toks_ref, bik_ref, t2k_ref):
    toks = toks_ref[...]
    bi = toks[:, :-1] * _V + toks[:, 1:]
    bik_ref[...] = jnp.concatenate(
        [bi, jnp.full((_B, 9), _DUMP, jnp.int32)], axis=1)
    t2k_ref[...] = jnp.concatenate(
        [toks[:, 2:], jnp.zeros((_B, 10), jnp.int32)], axis=1)


def _v16(x):
    return jnp.full((16,), x, jnp.int32)


@functools.partial(
    pl.kernel, mesh=_mesh,
    out_type=[jax.ShapeDtypeStruct((_B, 16), jnp.int32),
              jax.ShapeDtypeStruct((_B, 16), jnp.int32),
              jax.ShapeDtypeStruct((_B, 16), jnp.int32)],
    scratch_types=[pltpu.VMEM_SHARED((_SH_WORDS,), jnp.int32)]
                  + [pltpu.VMEM((_NCH, 16), jnp.int32) for _ in range(10)]
                  + [pltpu.VMEM((16,), jnp.int32), pltpu.SemaphoreType.DMA])
def _sc_distinct(bik_hbm, t2k_hbm, pbi_hbm, ptri_hbm, ppr_hbm,
                 shared,
                 key0, key1, t20, t21, ki0, ki1,
                 v0, v1, g0, g1, acc_v, sem):
    w = lax.axis_index("s")
    rows = (2 * w, 2 * w + 1)
    keys = (key0, key1)
    t2s = (t20, t21)
    kis = (ki0, ki1)
    vs = (v0, v1)
    gs = (g0, g1)
    iota = lax.iota(jnp.int32, 16)
    one = jnp.ones((16,), jnp.int32)
    zero = jnp.zeros((16,), jnp.int32)

    def scatter_all():
        cps = []
        for i in (0, 1):
            for c in range(_NCH):
                cps.append(pltpu.async_copy(vs[i].at[c],
                                            shared.at[kis[i].at[c]], sem))
        for cp in cps:
            cp.wait()
        plsc.subcore_barrier()

    def gather_all():
        cps = []
        for i in (0, 1):
            for c in range(_NCH):
                cps.append(pltpu.async_copy(shared.at[kis[i].at[c]],
                                            gs[i].at[c], sem))
        for cp in cps:
            cp.wait()

    for i in (0, 1):
        pltpu.sync_copy(bik_hbm.at[rows[i]], keys[i])
        pltpu.sync_copy(t2k_hbm.at[rows[i]], t2s[i])

    for i in (0, 1):
        for c in range(_NCH):
            kis[i][c, :] = keys[i][c, :]
            vs[i][c, :] = rows[i] * _NBI + c * 16 + iota
    scatter_all()
    gather_all()
    for i in (0, 1):
        acc = zero
        for c in range(_NCH):
            s_c = c * 16 + iota
            g_c = gs[i][c, :]
            j_c = rows[i] * _NBI + s_c
            acc = acc + jnp.where((s_c < _NBI) & (g_c == j_c), one, zero)
            kis[i][c, :] = jnp.where(s_c < _NTRI,
                                     g_c * 8 + (t2s[i][c, :] >> 7), _v16(_DUMP))
            vs[i][c, :] = rows[i] * _NTRI + s_c
        acc_v[...] = acc
        pltpu.sync_copy(acc_v, pbi_hbm.at[rows[i]])
    plsc.subcore_barrier()

    scatter_all()
    gather_all()
    for i in (0, 1):
        for c in range(_NCH):
            s_c = c * 16 + iota
            q_c = gs[i][c, :]
            kis[i][c, :] = jnp.where(s_c < _NTRI,
                                     q_c * 128 + (t2s[i][c, :] & 127), _v16(_DUMP))
    plsc.subcore_barrier()

    scatter_all()
    gather_all()
    for i in (0, 1):
        acc = zero
        for c in range(_NCH):
            s_c = c * 16 + iota
            h_c = gs[i][c, :]
            j3_c = rows[i] * _NTRI + s_c
            acc = acc + jnp.where((s_c < _NTRI) & (h_c == j3_c), one, zero)
            kis[i][c, :] = jnp.where(s_c < _NTRI,
                                     rows[i] * (_B * _NTRI) + h_c, _v16(_DUMP))
        acc_v[...] = acc
        pltpu.sync_copy(acc_v, ptri_hbm.at[rows[i]])
    plsc.subcore_barrier()

    scatter_all()
    gather_all()
    for i in (0, 1):
        acc = zero
        for c in range(_NCH):
            s_c = c * 16 + iota
            j3_c = rows[i] * _NTRI + s_c
            acc = acc + jnp.where((s_c < _NTRI) & (gs[i][c, :] == j3_c), one, zero)
        acc_v[...] = acc
        pltpu.sync_copy(acc_v, ppr_hbm.at[rows[i]])


@jax.jit
def _run(toks, logits):
    toks = toks.astype(jnp.int32)
    lg2 = logits.reshape(_B * _S, _V)
    rows = 1600

    bik, t2k = pl.pallas_call(
        _prep_body,
        out_shape=[jax.ShapeDtypeStruct((_B, 208), jnp.int32),
                   jax.ShapeDtypeStruct((_B, 208), jnp.int32)],
    )(toks)
    bik3 = bik.reshape(_B, _NCH, 16)
    t2k3 = t2k.reshape(_B, _NCH, 16)

    part_bi, part_tri, part_pr = _sc_distinct(bik3, t2k3)

    conf, counts, pres = pl.pallas_call(
        _conf_hist_body,
        grid=(_B * _S // rows,),
        in_specs=[pl.BlockSpec((rows, _V), lambda i: (i, 0)),
                  pl.BlockSpec((_B, _S), lambda i: (0, 0))],
        out_specs=[pl.BlockSpec((1, 1), lambda i: (0, 0)),
                   pl.BlockSpec((1, _V), lambda i: (0, 0)),
                   pl.BlockSpec((_B, _V), lambda i: (0, 0))],
        out_shape=[jax.ShapeDtypeStruct((1, 1), jnp.float32),
                   jax.ShapeDtypeStruct((1, _V), jnp.float32),
                   jax.ShapeDtypeStruct((_B, _V), jnp.float32)],
    )(lg2, toks)

    out = pl.pallas_call(
        _stats_body,
        out_shape=jax.ShapeDtypeStruct((1, 6), jnp.float32),
    )(counts, pres, part_bi, part_tri, part_pr, conf)
    return out.reshape(6)


def kernel(generated_tokens, generated_logits, vocab_size):
    return _run(generated_tokens, generated_logits)
